# Initial kernel scaffold; baseline (speedup 1.0000x reference)
import jax
import jax.numpy as jnp
from jax.experimental import pallas as pl


def kernel(vertices, template):
    raise NotImplementedError("write your pallas kernel here")


# speedup vs baseline: 1.0270x; 1.0270x over previous
import jax
import jax.numpy as jnp
from jax.experimental import pallas as pl
from jax.experimental.pallas import tpu as pltpu

V = 2048
K = 8
TOPK = 9
ROWS = 256


def _knn_body(vcol_ref, vrow_ref, out_ref):
    xi = vcol_ref[:, 0:1]
    yi = vcol_ref[:, 1:2]
    zi = vcol_ref[:, 2:3]
    xj = vrow_ref[0:1, :]
    yj = vrow_ref[1:2, :]
    zj = vrow_ref[2:3, :]
    dx = xi - xj
    dy = yi - yj
    dz = zi - zj
    sq = (dx * dx + dy * dy) + dz * dz
    dist = jnp.sqrt(jnp.maximum(sq, 1e-12))
    idxs = jax.lax.broadcasted_iota(jnp.int32, dist.shape, 1)
    out_ref[...] = jnp.zeros((ROWS, 16), jnp.int32)
    d = dist
    big = jnp.float32(jnp.inf)
    for t in range(TOPK):
        m = jnp.min(d, axis=1, keepdims=True)
        cand = jnp.where(d == m, idxs, V)
        j = jnp.min(cand, axis=1, keepdims=True)
        out_ref[:, t : t + 1] = j
        d = jnp.where(idxs == j, big, d)


def _knn(v):
    vrow = v.T
    return pl.pallas_call(
        _knn_body,
        grid=(V // ROWS,),
        in_specs=[
            pl.BlockSpec((ROWS, 3), lambda i: (i, 0)),
            pl.BlockSpec((3, V), lambda i: (0, 0)),
        ],
        out_specs=pl.BlockSpec((ROWS, 16), lambda i: (i, 0)),
        out_shape=jax.ShapeDtypeStruct((V, 16), jnp.int32),
    )(v, vrow)


def _compute_bc(template, projections):
    Vn, Kn = projections.shape[0], projections.shape[1]
    diff = template[None, :, :, None, :] - projections[:, None, None, :, :]
    dists = jnp.sqrt(jnp.maximum(jnp.sum(diff * diff, axis=-1), 1e-12))
    hierarchy = jnp.argsort(dists, axis=-1)
    iv = jnp.arange(Vn)[:, None, None, None]
    other_proj = projections[iv, hierarchy]
    closest_proj = other_proj[:, :, :, 0:1, :]
    v0 = other_proj - closest_proj
    v1 = v0
    v2 = template[None, :, :, None, :] - closest_proj
    v2s = v2[:, :, :, 0, :]
    dot00 = jnp.einsum('vrani,vrani->vran', v0, v0)
    dot01 = jnp.einsum('vrani,vrami->vranm', v0, v1)
    dot02 = jnp.einsum('vrani,vrai->vran', v0, v2s)
    dot11 = jnp.einsum('vrani,vrani->vran', v1, v1)
    dot12 = jnp.einsum('vrani,vrai->vran', v1, v2s)
    den = jnp.einsum('vran,vram->vranm', dot00, dot11) - dot01 * dot01
    den = jnp.where(den == 0.0, 1e-10, den)
    p2 = (jnp.einsum('vram,vran->vranm', dot11, dot02) - jnp.einsum('vranm,vram->vranm', dot01, dot12)) / den
    p1 = (jnp.einsum('vran,vram->vranm', dot00, dot12) - jnp.einsum('vranm,vran->vranm', dot01, dot02)) / den
    p0 = 1.0 - p2 - p1
    weights = jnp.stack([p0, p2, p1], axis=-1)
    score = jnp.max(jnp.abs(weights), axis=-1) + jnp.sum(jax.nn.relu(-weights), axis=-1)
    R, A = template.shape[0], template.shape[1]
    flat = score.reshape(Vn, R, A, Kn * Kn)
    amin = jnp.argmin(flat, axis=-1)
    row = amin // Kn
    col = amin % Kn
    iv3 = jnp.arange(Vn)[:, None, None]
    ir = jnp.arange(R)[None, :, None]
    ia = jnp.arange(A)[None, None, :]
    w_sel = weights[iv3, ir, ia, row, col]
    bc_idx = jnp.stack([row, col], axis=-1)
    bc_proj = jnp.take_along_axis(hierarchy, bc_idx, axis=3)
    bc_indices = jnp.concatenate([hierarchy[:, :, :, 0:1], bc_proj], axis=-1).astype(jnp.int32)
    return w_sel, bc_indices


def _per_batch(v, template):
    nbr16 = _knn(v)
    nbr_idx = nbr16[:, :K]
    idx9 = nbr16[:, K]
    neighborhoods = v[nbr_idx] - v[:, None, :]
    d = jnp.sqrt(jnp.maximum(jnp.sum(neighborhoods * neighborhoods, axis=-1), 1e-12))
    rdiff = v[idx9] - v
    rx, ry, rz = rdiff[:, 0], rdiff[:, 1], rdiff[:, 2]
    radii = jnp.sqrt(jnp.maximum((rx * rx + ry * ry) + rz * rz, 1e-12))
    w = jax.nn.relu(radii[:, None] - d)
    cov = jnp.einsum('vk,vki,vkj->vij', w, neighborhoods, neighborhoods) / (
        jnp.sum(w, axis=-1)[:, None, None] + 1e-10)
    evals, evecs = jnp.linalg.eigh(cov)
    lrf = evecs[:, :, ::-1]
    coords = jnp.einsum('vki,vij->vkj', neighborhoods, lrf)
    tangent = coords[:, :, :2]
    n2 = jnp.sqrt(jnp.maximum(jnp.sum(tangent * tangent, axis=-1), 1e-12))
    projections = tangent * (d / n2)[:, :, None]
    w_bc, bc_idx = _compute_bc(template, projections)
    proj_indices = nbr_idx[jnp.arange(V)[:, None, None, None], bc_idx].astype(jnp.float32)
    return jnp.stack([proj_indices, w_bc], axis=-1)


def kernel(vertices, template):
    return jax.vmap(lambda v: _per_batch(v, template))(vertices)
